# Initial kernel scaffold; baseline (speedup 1.0000x reference)
import functools

import jax
import jax.numpy as jnp
from jax import lax
from jax.experimental import pallas as pl
from jax.experimental.pallas import tpu as pltpu
from jax.experimental.pallas import tpu_sc as plsc

HIDDEN = 64
POLICY_DIM = 7
D_FEAT = 128
BATCH_SIZE = 64
N_PIECE = 5000
N_CELL = 5000
N_EDGES = 320000
N_NODES = N_PIECE + N_CELL


def _leaky(x):
    return jnp.where(x >= 0, x, 0.01 * x)


def _enc_one(f, W, b, g, be):
    h = jnp.dot(f, W, preferred_element_type=jnp.float32) + b
    mu = jnp.mean(h, axis=-1, keepdims=True)
    var = jnp.mean((h - mu) * (h - mu), axis=-1, keepdims=True)
    h = (h - mu) * lax.rsqrt(var + 1e-5) * g + be
    return _leaky(h)


def _stage_a_body(fp_ref, fc_ref, batch_ref,
                  Wp_ref, bp_ref, gp_ref, bep_ref,
                  Wc_ref, bc_ref, gc_ref, bec_ref,
                  Wc1_ref, bc1_ref, Wc2_ref, bc2_ref,
                  x_out, critic_out):
    hp = _enc_one(fp_ref[...], Wp_ref[...], bp_ref[...], gp_ref[...], bep_ref[...])
    hc = _enc_one(fc_ref[...], Wc_ref[...], bc_ref[...], gc_ref[...], bec_ref[...])
    x_out[0:N_PIECE, :] = hp
    x_out[N_PIECE:N_NODES, :] = hc
    x_all = jnp.concatenate([hp, hc], axis=0)
    b_row = batch_ref[...]
    seg_ids = lax.broadcasted_iota(jnp.int32, (BATCH_SIZE, N_NODES), 0)
    onehot = (seg_ids == jnp.broadcast_to(b_row, (BATCH_SIZE, N_NODES))).astype(jnp.float32)
    seg_sum = jnp.dot(onehot, x_all, preferred_element_type=jnp.float32)
    cnt = jnp.sum(onehot, axis=1, keepdims=True)
    pooled = seg_sum / jnp.maximum(cnt, 1.0)
    c1 = _leaky(jnp.dot(pooled, Wc1_ref[...], preferred_element_type=jnp.float32) + bc1_ref[...])
    critic_out[...] = jnp.dot(c1, Wc2_ref[...], preferred_element_type=jnp.float32) + bc2_ref[...]


_NW = 32
_EPW = N_EDGES // _NW
_CH = 1000
_NCHUNK = _EPW // _CH


def _gather_body(x_hbm, src_hbm, dst_hbm, srcg_hbm, dstg_hbm,
                 idx_v, rows_v, sem):
    wid = lax.axis_index("s") * 2 + lax.axis_index("c")
    base = wid * _EPW

    def chunk(k, carry):
        off = base + k * _CH
        pltpu.sync_copy(src_hbm.at[pl.ds(off, _CH)], idx_v)
        pltpu.async_copy(x_hbm.at[idx_v], rows_v, sem).wait()
        pltpu.sync_copy(rows_v, srcg_hbm.at[pl.ds(off, _CH)])
        pltpu.sync_copy(dst_hbm.at[pl.ds(off, _CH)], idx_v)
        pltpu.async_copy(x_hbm.at[idx_v], rows_v, sem).wait()
        pltpu.sync_copy(rows_v, dstg_hbm.at[pl.ds(off, _CH)])
        return carry

    lax.fori_loop(0, _NCHUNK, chunk, 0)


def _make_gather():
    mesh = plsc.VectorSubcoreMesh(core_axis_name="c", subcore_axis_name="s")
    return functools.partial(
        pl.kernel,
        mesh=mesh,
        out_type=[
            jax.ShapeDtypeStruct((N_EDGES, HIDDEN), jnp.float32),
            jax.ShapeDtypeStruct((N_EDGES, HIDDEN), jnp.float32),
        ],
        scratch_types=[
            pltpu.VMEM((_CH,), jnp.int32),
            pltpu.VMEM((_CH, HIDDEN), jnp.float32),
            pltpu.SemaphoreType.DMA,
        ],
    )(_gather_body)


_ROWS = 2560


def _stage_c_body(src_ref, dst_ref, W1_ref, b1_ref, W2_ref, b2_ref, out_ref):
    pair = src_ref[...] * dst_ref[...]
    h = _leaky(jnp.dot(pair, W1_ref[...], preferred_element_type=jnp.float32) + b1_ref[...])
    out_ref[...] = jnp.dot(h, W2_ref[...], preferred_element_type=jnp.float32) + b2_ref[...]


def kernel(feat_piece, feat_cell, edge_src, edge_dst, batch,
           W_piece, b_piece, g_piece, be_piece,
           W_cell, b_cell, g_cell, be_cell,
           Wp1, bp1, Wp2, bp2, Wc1, bc1, Wc2, bc2):
    r1 = lambda v: v.reshape(1, -1)
    Wc2p = jnp.pad(Wc2, ((0, 0), (0, 7)))
    bc2p = jnp.pad(r1(bc2), ((0, 0), (0, 7)))
    Wp2p = jnp.pad(Wp2, ((0, 0), (0, 1)))
    bp2p = jnp.pad(r1(bp2), ((0, 0), (0, 1)))

    x_all, critic_pad = pl.pallas_call(
        _stage_a_body,
        out_shape=[
            jax.ShapeDtypeStruct((N_NODES, HIDDEN), jnp.float32),
            jax.ShapeDtypeStruct((BATCH_SIZE, 8), jnp.float32),
        ],
    )(feat_piece, feat_cell, r1(batch),
      W_piece, r1(b_piece), r1(g_piece), r1(be_piece),
      W_cell, r1(b_cell), r1(g_cell), r1(be_cell),
      Wc1, r1(bc1), Wc2p, bc2p)

    src_g, dst_g = _make_gather()(x_all, edge_src, edge_dst)

    grid = N_EDGES // _ROWS
    row_spec = pl.BlockSpec((_ROWS, HIDDEN), lambda i: (i, 0))
    w_spec = lambda s: pl.BlockSpec(s, lambda i: (0, 0))
    logits_pad = pl.pallas_call(
        _stage_c_body,
        grid=(grid,),
        in_specs=[
            row_spec, row_spec,
            w_spec((HIDDEN, HIDDEN)), w_spec((1, HIDDEN)),
            w_spec((HIDDEN, 8)), w_spec((1, 8)),
        ],
        out_specs=pl.BlockSpec((_ROWS, 8), lambda i: (i, 0)),
        out_shape=jax.ShapeDtypeStruct((N_EDGES, 8), jnp.float32),
    )(src_g, dst_g, Wp1, r1(bp1), Wp2p, bp2p)

    return (logits_pad[:, :POLICY_DIM], critic_pad[:, :1])


# speedup vs baseline: 2.2664x; 2.2664x over previous
import functools

import jax
import jax.numpy as jnp
from jax import lax
from jax.experimental import pallas as pl
from jax.experimental.pallas import tpu as pltpu
from jax.experimental.pallas import tpu_sc as plsc

HIDDEN = 64
POLICY_DIM = 7
D_FEAT = 128
BATCH_SIZE = 64
N_PIECE = 5000
N_CELL = 5000
N_EDGES = 320000
N_NODES = N_PIECE + N_CELL


def _leaky(x):
    return jnp.where(x >= 0, x, 0.01 * x)


def _enc_one(f, W, b, g, be):
    h = jnp.dot(f, W, preferred_element_type=jnp.float32) + b
    mu = jnp.mean(h, axis=-1, keepdims=True)
    var = jnp.mean((h - mu) * (h - mu), axis=-1, keepdims=True)
    h = (h - mu) * lax.rsqrt(var + 1e-5) * g + be
    return _leaky(h)


def _stage_a_body(fp_ref, fc_ref, batch_ref,
                  Wp_ref, bp_ref, gp_ref, bep_ref,
                  Wc_ref, bc_ref, gc_ref, bec_ref,
                  Wc1_ref, bc1_ref, Wc2_ref, bc2_ref,
                  x_out, critic_out):
    hp = _enc_one(fp_ref[...], Wp_ref[...], bp_ref[...], gp_ref[...], bep_ref[...])
    hc = _enc_one(fc_ref[...], Wc_ref[...], bc_ref[...], gc_ref[...], bec_ref[...])
    x_out[0:N_PIECE, :] = hp
    x_out[N_PIECE:N_NODES, :] = hc
    x_all = jnp.concatenate([hp, hc], axis=0)
    b_row = batch_ref[...]
    seg_ids = lax.broadcasted_iota(jnp.int32, (BATCH_SIZE, N_NODES), 0)
    onehot = (seg_ids == jnp.broadcast_to(b_row, (BATCH_SIZE, N_NODES))).astype(jnp.float32)
    seg_sum = jnp.dot(onehot, x_all, preferred_element_type=jnp.float32)
    cnt = jnp.sum(onehot, axis=1, keepdims=True)
    pooled = seg_sum / jnp.maximum(cnt, 1.0)
    c1 = _leaky(jnp.dot(pooled, Wc1_ref[...], preferred_element_type=jnp.float32) + bc1_ref[...])
    critic_out[...] = jnp.dot(c1, Wc2_ref[...], preferred_element_type=jnp.float32) + bc2_ref[...]


_NW = 32
_EPW = N_EDGES // _NW
_CH = 1000
_NCHUNK = _EPW // _CH


def _gather_body(x_hbm, src_hbm, dst_hbm, srcg_hbm, dstg_hbm,
                 idx_v, rows_v, sem):
    wid = lax.axis_index("s") * 2 + lax.axis_index("c")
    base = wid * _EPW

    def chunk(k, carry):
        off = base + k * _CH
        pltpu.sync_copy(src_hbm.at[pl.ds(off, _CH)], idx_v)
        pltpu.async_copy(x_hbm.at[idx_v], rows_v, sem).wait()
        pltpu.sync_copy(rows_v, srcg_hbm.at[pl.ds(off, _CH)])
        pltpu.sync_copy(dst_hbm.at[pl.ds(off, _CH)], idx_v)
        pltpu.async_copy(x_hbm.at[idx_v], rows_v, sem).wait()
        pltpu.sync_copy(rows_v, dstg_hbm.at[pl.ds(off, _CH)])
        return carry

    lax.fori_loop(0, _NCHUNK, chunk, 0)


def _make_gather():
    mesh = plsc.VectorSubcoreMesh(core_axis_name="c", subcore_axis_name="s")
    return functools.partial(
        pl.kernel,
        mesh=mesh,
        out_type=[
            jax.ShapeDtypeStruct((N_EDGES, HIDDEN), jnp.float32),
            jax.ShapeDtypeStruct((N_EDGES, HIDDEN), jnp.float32),
        ],
        scratch_types=[
            pltpu.VMEM((_CH,), jnp.int32),
            pltpu.VMEM((_CH, HIDDEN), jnp.float32),
            pltpu.SemaphoreType.DMA,
        ],
        compiler_params=pltpu.CompilerParams(use_tc_tiling_on_sc=False),
    )(_gather_body)


_ROWS = 2560


def _stage_c_body(src_ref, dst_ref, W1_ref, b1_ref, W2_ref, b2_ref, out_ref):
    pair = src_ref[...] * dst_ref[...]
    h = _leaky(jnp.dot(pair, W1_ref[...], preferred_element_type=jnp.float32) + b1_ref[...])
    out_ref[...] = jnp.dot(h, W2_ref[...], preferred_element_type=jnp.float32) + b2_ref[...]


def kernel(feat_piece, feat_cell, edge_src, edge_dst, batch,
           W_piece, b_piece, g_piece, be_piece,
           W_cell, b_cell, g_cell, be_cell,
           Wp1, bp1, Wp2, bp2, Wc1, bc1, Wc2, bc2):
    r1 = lambda v: v.reshape(1, -1)
    Wc2p = jnp.pad(Wc2, ((0, 0), (0, 7)))
    bc2p = jnp.pad(r1(bc2), ((0, 0), (0, 7)))
    Wp2p = jnp.pad(Wp2, ((0, 0), (0, 1)))
    bp2p = jnp.pad(r1(bp2), ((0, 0), (0, 1)))

    x_all, critic_pad = pl.pallas_call(
        _stage_a_body,
        out_shape=[
            jax.ShapeDtypeStruct((N_NODES, HIDDEN), jnp.float32),
            jax.ShapeDtypeStruct((BATCH_SIZE, 8), jnp.float32),
        ],
    )(feat_piece, feat_cell, r1(batch),
      W_piece, r1(b_piece), r1(g_piece), r1(be_piece),
      W_cell, r1(b_cell), r1(g_cell), r1(be_cell),
      Wc1, r1(bc1), Wc2p, bc2p)

    src_g, dst_g = _make_gather()(x_all, edge_src, edge_dst)

    grid = N_EDGES // _ROWS
    row_spec = pl.BlockSpec((_ROWS, HIDDEN), lambda i: (i, 0))
    w_spec = lambda s: pl.BlockSpec(s, lambda i: (0, 0))
    logits_pad = pl.pallas_call(
        _stage_c_body,
        grid=(grid,),
        in_specs=[
            row_spec, row_spec,
            w_spec((HIDDEN, HIDDEN)), w_spec((1, HIDDEN)),
            w_spec((HIDDEN, 8)), w_spec((1, 8)),
        ],
        out_specs=pl.BlockSpec((_ROWS, 8), lambda i: (i, 0)),
        out_shape=jax.ShapeDtypeStruct((N_EDGES, 8), jnp.float32),
    )(src_g, dst_g, Wp1, r1(bp1), Wp2p, bp2p)

    return (logits_pad[:, :POLICY_DIM], critic_pad[:, :1])


# speedup vs baseline: 3.1132x; 1.3736x over previous
import functools

import jax
import jax.numpy as jnp
from jax import lax
from jax.experimental import pallas as pl
from jax.experimental.pallas import tpu as pltpu
from jax.experimental.pallas import tpu_sc as plsc

HIDDEN = 64
POLICY_DIM = 7
D_FEAT = 128
BATCH_SIZE = 64
N_PIECE = 5000
N_CELL = 5000
N_EDGES = 320000
N_NODES = N_PIECE + N_CELL


def _leaky(x):
    return jnp.where(x >= 0, x, 0.01 * x)


def _ln_leaky(h, g, be):
    hL, hR = h[:, :HIDDEN], h[:, HIDDEN:]

    def one(side):
        mu = jnp.mean(side, axis=-1, keepdims=True)
        var = jnp.mean((side - mu) * (side - mu), axis=-1, keepdims=True)
        return (side - mu) * lax.rsqrt(var + 1e-5)

    hn = jnp.concatenate([one(hL), one(hR)], axis=1)
    return _leaky(hn * g + be)


def _stage_a_body(fp_ref, fc_ref, be_ref, bo_ref,
                  Wp_ref, bp_ref, gp_ref, bep_ref,
                  Wc_ref, bc_ref, gc_ref, bec_ref,
                  Wc1_ref, bc1_ref, Wc2_ref, bc2_ref,
                  x_out, critic_out):
    hp = _ln_leaky(jnp.dot(fp_ref[...], Wp_ref[...], preferred_element_type=jnp.float32) + bp_ref[...],
                   gp_ref[...], bep_ref[...])
    hc = _ln_leaky(jnp.dot(fc_ref[...], Wc_ref[...], preferred_element_type=jnp.float32) + bc_ref[...],
                   gc_ref[...], bec_ref[...])
    x_out[0:N_PIECE // 2, :] = hp
    x_out[N_PIECE // 2:N_NODES // 2, :] = hc
    xpk = jnp.concatenate([hp, hc], axis=0)
    half = N_NODES // 2
    seg_ids = lax.broadcasted_iota(jnp.int32, (BATCH_SIZE, half), 0)
    ohE = (seg_ids == jnp.broadcast_to(be_ref[...], (BATCH_SIZE, half))).astype(jnp.float32)
    ohO = (seg_ids == jnp.broadcast_to(bo_ref[...], (BATCH_SIZE, half))).astype(jnp.float32)
    seg_sum = (jnp.dot(ohE, xpk[:, :HIDDEN], preferred_element_type=jnp.float32)
               + jnp.dot(ohO, xpk[:, HIDDEN:], preferred_element_type=jnp.float32))
    cnt = jnp.sum(ohE, axis=1, keepdims=True) + jnp.sum(ohO, axis=1, keepdims=True)
    pooled = seg_sum / jnp.maximum(cnt, 1.0)
    c1 = _leaky(jnp.dot(pooled, Wc1_ref[...], preferred_element_type=jnp.float32) + bc1_ref[...])
    critic_out[...] = jnp.dot(c1, Wc2_ref[...], preferred_element_type=jnp.float32) + bc2_ref[...]


_NW = 32
_EPW = N_EDGES // _NW
_CH = 1000
_NCHUNK = _EPW // _CH


def _gather_body(x_hbm, src_hbm, dst_hbm, srcg_hbm, dstg_hbm,
                 idx_v, rows_v, sem):
    wid = lax.axis_index("s") * 2 + lax.axis_index("c")
    base = wid * _EPW

    def chunk(k, carry):
        off = base + k * _CH
        pltpu.sync_copy(src_hbm.at[pl.ds(off, _CH)], idx_v)
        pltpu.async_copy(x_hbm.at[idx_v], rows_v, sem).wait()
        pltpu.sync_copy(rows_v, srcg_hbm.at[pl.ds(off, _CH)])
        pltpu.sync_copy(dst_hbm.at[pl.ds(off, _CH)], idx_v)
        pltpu.async_copy(x_hbm.at[idx_v], rows_v, sem).wait()
        pltpu.sync_copy(rows_v, dstg_hbm.at[pl.ds(off, _CH)])
        return carry

    lax.fori_loop(0, _NCHUNK, chunk, 0)


def _make_gather():
    mesh = plsc.VectorSubcoreMesh(core_axis_name="c", subcore_axis_name="s")
    return functools.partial(
        pl.kernel,
        mesh=mesh,
        out_type=[
            jax.ShapeDtypeStruct((N_EDGES, HIDDEN), jnp.float32),
            jax.ShapeDtypeStruct((N_EDGES, HIDDEN), jnp.float32),
        ],
        scratch_types=[
            pltpu.VMEM((_CH,), jnp.int32),
            pltpu.VMEM((_CH, HIDDEN), jnp.float32),
            pltpu.SemaphoreType.DMA,
        ],
        compiler_params=pltpu.CompilerParams(use_tc_tiling_on_sc=False),
    )(_gather_body)


_ROWS = 1600


def _stage_c_body(src_ref, dst_ref, W1_ref, b1_ref, W2_ref, b2_ref, out_ref):
    pair = src_ref[...] * dst_ref[...]
    h = _leaky(jnp.dot(pair, W1_ref[...], preferred_element_type=jnp.float32) + b1_ref[...])
    out_ref[...] = jnp.dot(h, W2_ref[...], preferred_element_type=jnp.float32) + b2_ref[...]


def kernel(feat_piece, feat_cell, edge_src, edge_dst, batch,
           W_piece, b_piece, g_piece, be_piece,
           W_cell, b_cell, g_cell, be_cell,
           Wp1, bp1, Wp2, bp2, Wc1, bc1, Wc2, bc2):
    r1 = lambda v: v.reshape(1, -1)
    Wc2p = jnp.pad(Wc2, ((0, 0), (0, 7)))
    bc2p = jnp.pad(r1(bc2), ((0, 0), (0, 7)))
    Wp2p = jnp.pad(Wp2, ((0, 0), (0, 1)))
    bp2p = jnp.pad(r1(bp2), ((0, 0), (0, 1)))
    z = jnp.zeros((HIDDEN, HIDDEN), jnp.float32)
    W1b = jnp.block([[Wp1, z], [z, Wp1]])
    b1b = jnp.concatenate([r1(bp1), r1(bp1)], axis=1)
    z2 = jnp.zeros((HIDDEN, 8), jnp.float32)
    W2b = jnp.block([[Wp2p, z2], [z2, Wp2p]])
    b2b = jnp.concatenate([bp2p, bp2p], axis=1)

    zf = jnp.zeros((D_FEAT, HIDDEN), jnp.float32)
    Wpb = jnp.block([[W_piece, zf], [zf, W_piece]])
    Wcb = jnp.block([[W_cell, zf], [zf, W_cell]])
    dup = lambda v: jnp.concatenate([r1(v), r1(v)], axis=1)
    batch_even = batch[0::2].reshape(1, -1)
    batch_odd = batch[1::2].reshape(1, -1)

    x2, critic_pad = pl.pallas_call(
        _stage_a_body,
        out_shape=[
            jax.ShapeDtypeStruct((N_NODES // 2, 2 * HIDDEN), jnp.float32),
            jax.ShapeDtypeStruct((BATCH_SIZE, 8), jnp.float32),
        ],
    )(feat_piece.reshape(N_PIECE // 2, 2 * D_FEAT),
      feat_cell.reshape(N_CELL // 2, 2 * D_FEAT),
      batch_even, batch_odd,
      Wpb, dup(b_piece), dup(g_piece), dup(be_piece),
      Wcb, dup(b_cell), dup(g_cell), dup(be_cell),
      Wc1, r1(bc1), Wc2p, bc2p)

    x_all = x2.reshape(N_NODES, HIDDEN)
    src_g, dst_g = _make_gather()(x_all, edge_src, edge_dst)

    src2 = src_g.reshape(N_EDGES // 2, 2 * HIDDEN)
    dst2 = dst_g.reshape(N_EDGES // 2, 2 * HIDDEN)
    grid = (N_EDGES // 2) // _ROWS
    row_spec = pl.BlockSpec((_ROWS, 2 * HIDDEN), lambda i: (i, 0))
    w_spec = lambda s: pl.BlockSpec(s, lambda i: (0, 0))
    logits_pad = pl.pallas_call(
        _stage_c_body,
        grid=(grid,),
        in_specs=[
            row_spec, row_spec,
            w_spec((2 * HIDDEN, 2 * HIDDEN)), w_spec((1, 2 * HIDDEN)),
            w_spec((2 * HIDDEN, 16)), w_spec((1, 16)),
        ],
        out_specs=pl.BlockSpec((_ROWS, 16), lambda i: (i, 0)),
        out_shape=jax.ShapeDtypeStruct((N_EDGES // 2, 16), jnp.float32),
    )(src2, dst2, W1b, b1b, W2b, b2b)

    return (logits_pad.reshape(N_EDGES, 8)[:, :POLICY_DIM], critic_pad[:, :1])
